# baseline (speedup 1.0000x reference)
import functools

import jax
import jax.numpy as jnp
from jax import lax
from jax.experimental import pallas as pl
from jax.experimental.pallas import tpu as pltpu
from jax.experimental.pallas import tpu_sc as plsc

NC = 2
NS = 16
LANES = 16
NW = NC * NS

CH = 80


def _sc_spmm(emb, pk, zeros):
    N, D = emb.shape
    n_chunks = pk.shape[1]
    Np = zeros.shape[0]
    rows_per_tile = Np // NS
    assert n_chunks % 2 == 1 and n_chunks >= 3

    mesh = plsc.VectorSubcoreMesh(core_axis_name="c", subcore_axis_name="s")

    @functools.partial(
        pl.kernel,
        out_type=jax.ShapeDtypeStruct((NC, Np, D), jnp.float32),
        mesh=mesh,
        compiler_params=pltpu.CompilerParams(needs_layout_passes=False,
                                             use_tc_tiling_on_sc=False),
        scratch_types=[
            pltpu.VMEM_SHARED((Np, D), jnp.float32),
            pltpu.VMEM((3, CH), jnp.int32),
            pltpu.VMEM((3, CH), jnp.int32),
            pltpu.VMEM((CH, D), jnp.float32),
            pltpu.VMEM((CH, D), jnp.float32),
            pltpu.SemaphoreType.DMA,
            pltpu.SemaphoreType.DMA,
            pltpu.SemaphoreType.DMA,
            pltpu.SemaphoreType.DMA,
        ],
    )
    def spmm(emb_hbm, pk_hbm, zeros_hbm, parts_hbm,
             agg_sh, eb0, eb1, rows0, rows1, es0, es1, gs0, gs1):
        c = lax.axis_index("c")
        s = lax.axis_index("s")
        wid = s * NC + c
        r0 = s * rows_per_tile
        pltpu.sync_copy(zeros_hbm.at[pl.ds(r0, rows_per_tile)],
                        agg_sh.at[pl.ds(r0, rows_per_tile)])
        plsc.subcore_barrier()

        ebufs = (eb0, eb1)
        esems = (es0, es1)
        rows = (rows0, rows1)
        gsems = (gs0, gs1)

        def issue_gather(b):
            pltpu.async_copy(emb_hbm.at[ebufs[b].at[0]], rows[b], gsems[b])

        def wait_gather(b):
            pltpu.make_async_copy(emb_hbm.at[ebufs[b].at[0]],
                                  rows[b], gsems[b]).wait()

        def wait_eload(b):
            pltpu.make_async_copy(pk_hbm.at[wid, 0], ebufs[b],
                                  esems[b]).wait()

        def mul_chunk(b):
            rbuf = rows[b]
            wrow = ebufs[b]

            def group_body(g, _):
                w16 = lax.bitcast_convert_type(
                    wrow[2, pl.ds(g * LANES, LANES)], jnp.float32)
                for el in range(LANES):
                    wb = w16[el]
                    e = g * LANES + el
                    for k in range(D // LANES):
                        sl = pl.ds(k * LANES, LANES)
                        rbuf[e, sl] = rbuf[e, sl] * wb
                return 0

            lax.fori_loop(0, CH // LANES, group_body, 0)

        def scatter_chunk(b):
            pltpu.sync_copy(rows[b], agg_sh.at[ebufs[b].at[1]], add=True)

        def iteration(i, b, b1):
            wait_eload(b1)
            issue_gather(b1)
            wait_gather(b)
            mul_chunk(b)
            scatter_chunk(b)
            pltpu.async_copy(pk_hbm.at[wid, jnp.minimum(i + 2, n_chunks - 1)],
                             ebufs[b], esems[b])

        pltpu.sync_copy(pk_hbm.at[wid, 0], eb0)
        pltpu.async_copy(pk_hbm.at[wid, 1], eb1, es1)
        issue_gather(0)

        @pl.loop(0, n_chunks - 1, step=2)
        def _(t):
            iteration(t, 0, 1)
            iteration(t + 1, 1, 0)

        wait_gather(0)
        wait_eload(1)
        mul_chunk(0)
        scatter_chunk(0)

        plsc.subcore_barrier()
        pltpu.sync_copy(agg_sh.at[pl.ds(r0, rows_per_tile)],
                        parts_hbm.at[c, pl.ds(r0, rows_per_tile)])

    return spmm(emb, pk, zeros)


def _tc_dense(emb, parts, W1, b1, W2, b2):
    N, D = emb.shape
    BM = 2000
    dn = (((1,), (1,)), ((), ()))

    def body(emb_ref, parts_ref, w1_ref, b1_ref, w2_ref, b2_ref, out_ref):
        x = emb_ref[...]
        agg = parts_ref[0] + parts_ref[1]
        w1 = w1_ref[...]
        w2 = w2_ref[...]
        b1v = b1_ref[...]
        b2v = b2_ref[...]
        self_emb = lax.dot_general(x, w1, dn, preferred_element_type=jnp.float32) + b1v
        neigh = lax.dot_general(agg, w2, dn, preferred_element_type=jnp.float32) + b2v
        inter = lax.dot_general(neigh * x, w2, dn,
                                preferred_element_type=jnp.float32) + b2v
        o = self_emb + neigh + inter
        out_ref[...] = jnp.where(o >= 0, o, 0.2 * o)

    return pl.pallas_call(
        body,
        grid=(N // BM,),
        in_specs=[
            pl.BlockSpec((BM, D), lambda i: (i, 0)),
            pl.BlockSpec((NC, BM, D), lambda i: (0, i, 0)),
            pl.BlockSpec((D, D), lambda i: (0, 0)),
            pl.BlockSpec((1, D), lambda i: (0, 0)),
            pl.BlockSpec((D, D), lambda i: (0, 0)),
            pl.BlockSpec((1, D), lambda i: (0, 0)),
        ],
        out_specs=pl.BlockSpec((BM, D), lambda i: (i, 0)),
        out_shape=jax.ShapeDtypeStruct((N, D), jnp.float32),
    )(emb, parts, W1, b1.reshape(1, D), W2, b2.reshape(1, D))


def kernel(embeddings, adj_edge_index, adj_edge_weight, W1, b1, W2, b2):
    N, D = embeddings.shape
    E = adj_edge_index.shape[1]
    epw = E // NW
    Np = -(-N // (8 * NS)) * (8 * NS)
    n_chunks = -(-epw // CH)
    if n_chunks % 2 == 0:
        n_chunks += 1
    pad = n_chunks * CH - epw
    src2 = adj_edge_index[0].reshape(NW, epw)
    dst2 = adj_edge_index[1].reshape(NW, epw)
    w2 = adj_edge_weight.reshape(NW, epw)
    if pad:
        psrc = jnp.broadcast_to(jnp.arange(pad, dtype=jnp.int32) % N, (NW, pad))
        pdst = jnp.broadcast_to(
            N + (jnp.arange(pad, dtype=jnp.int32) % (Np - N)), (NW, pad))
        src2 = jnp.concatenate([src2, psrc], axis=1)
        dst2 = jnp.concatenate([dst2, pdst], axis=1)
        w2 = jnp.concatenate([w2, jnp.zeros((NW, pad), w2.dtype)], axis=1)
    wbits = lax.bitcast_convert_type(w2, jnp.int32)
    pk = jnp.stack([src2.reshape(NW, n_chunks, CH),
                    dst2.reshape(NW, n_chunks, CH),
                    wbits.reshape(NW, n_chunks, CH)], axis=2)
    zeros = jnp.zeros((Np, D), jnp.float32)
    parts = _sc_spmm(embeddings, pk, zeros)
    return _tc_dense(embeddings, parts, W1, b1, W2, b2)


# speedup vs baseline: 1.4275x; 1.4275x over previous
import functools

import jax
import jax.numpy as jnp
from jax import lax
from jax.experimental import pallas as pl
from jax.experimental.pallas import tpu as pltpu
from jax.experimental.pallas import tpu_sc as plsc

NC = 2
NS = 16
LANES = 16
NW = NC * NS

CH = 80


def _sc_spmm(emb, src, dst, w, zeros):
    N, D = emb.shape
    E = src.shape[0]
    epw = E // NW
    n_chunks = epw // CH
    Np = zeros.shape[0]
    rows_per_tile = Np // NS

    mesh = plsc.VectorSubcoreMesh(core_axis_name="c", subcore_axis_name="s")

    assert n_chunks % 2 == 1

    @functools.partial(
        pl.kernel,
        out_type=jax.ShapeDtypeStruct((NC, Np, D), jnp.float32),
        mesh=mesh,
        compiler_params=pltpu.CompilerParams(needs_layout_passes=False,
                                             use_tc_tiling_on_sc=False),
        scratch_types=[
            pltpu.VMEM_SHARED((Np, D), jnp.float32),
            pltpu.VMEM((epw,), jnp.int32),
            pltpu.VMEM((epw,), jnp.float32),
            pltpu.VMEM((CH,), jnp.int32),
            pltpu.VMEM((CH,), jnp.int32),
            pltpu.VMEM((CH, D), jnp.float32),
            pltpu.VMEM((CH, D), jnp.float32),
            pltpu.SemaphoreType.DMA,
            pltpu.SemaphoreType.DMA,
            pltpu.SemaphoreType.DMA,
            pltpu.SemaphoreType.DMA,
        ],
    )
    def spmm(emb_hbm, src_hbm, dst_hbm, w_hbm, zeros_hbm, parts_hbm,
             agg_sh, src_v, w_v, dbuf0, dbuf1, rows0, rows1,
             gsem0, gsem1, dsem0, dsem1):
        c = lax.axis_index("c")
        s = lax.axis_index("s")
        wid = s * NC + c
        r0 = s * rows_per_tile
        pltpu.sync_copy(zeros_hbm.at[pl.ds(r0, rows_per_tile)],
                        agg_sh.at[pl.ds(r0, rows_per_tile)])
        pltpu.sync_copy(src_hbm.at[wid], src_v)
        pltpu.sync_copy(w_hbm.at[wid], w_v)
        plsc.subcore_barrier()

        rows = (rows0, rows1)
        gsems = (gsem0, gsem1)
        dbufs = (dbuf0, dbuf1)
        dsems = (dsem0, dsem1)

        def issue_chunk(i, b):
            pltpu.async_copy(dst_hbm.at[wid, i], dbufs[b], dsems[b])
            pltpu.async_copy(emb_hbm.at[src_v.at[pl.ds(i * CH, CH)]],
                             rows[b], gsems[b])

        def wait_gather(b):
            pltpu.make_async_copy(emb_hbm.at[src_v.at[pl.ds(0, CH)]],
                                  rows[b], gsems[b]).wait()

        def mul_chunk(i, b):
            rbuf = rows[b]

            def group_body(g, _):
                w16 = w_v[pl.ds(i * CH + g * LANES, LANES)]
                for el in range(LANES):
                    wb = w16[el]
                    e = g * LANES + el
                    for k in range(D // LANES):
                        sl = pl.ds(k * LANES, LANES)
                        rbuf[e, sl] = rbuf[e, sl] * wb
                return 0

            lax.fori_loop(0, CH // LANES, group_body, 0)

        def scatter_chunk(b):
            pltpu.make_async_copy(dst_hbm.at[wid, 0], dbufs[b], dsems[b]).wait()
            pltpu.sync_copy(rows[b], agg_sh.at[dbufs[b]], add=True)

        issue_chunk(0, 0)

        @pl.loop(0, n_chunks - 1, step=2)
        def _(t):
            issue_chunk(t + 1, 1)
            wait_gather(0)
            mul_chunk(t, 0)
            scatter_chunk(0)
            issue_chunk(t + 2, 0)
            wait_gather(1)
            mul_chunk(t + 1, 1)
            scatter_chunk(1)

        wait_gather(0)
        mul_chunk(n_chunks - 1, 0)
        scatter_chunk(0)

        plsc.subcore_barrier()
        pltpu.sync_copy(agg_sh.at[pl.ds(r0, rows_per_tile)],
                        parts_hbm.at[c, pl.ds(r0, rows_per_tile)])

    return spmm(emb, src.reshape(NW, epw), dst.reshape(NW, n_chunks, CH),
                w.reshape(NW, epw), zeros)


_DN = (((1,), (1,)), ((), ()))
_BM = 2000


def _tc_self(emb, W1, b1):
    N, D = emb.shape

    def body(emb_ref, w1_ref, b1_ref, out_ref):
        x = emb_ref[...]
        out_ref[...] = lax.dot_general(
            x, w1_ref[...], _DN, preferred_element_type=jnp.float32) + b1_ref[...]

    return pl.pallas_call(
        body,
        grid=(N // _BM,),
        in_specs=[
            pl.BlockSpec((_BM, D), lambda i: (i, 0)),
            pl.BlockSpec((D, D), lambda i: (0, 0)),
            pl.BlockSpec((1, D), lambda i: (0, 0)),
        ],
        out_specs=pl.BlockSpec((_BM, D), lambda i: (i, 0)),
        out_shape=jax.ShapeDtypeStruct((N, D), jnp.float32),
    )(emb, W1, b1.reshape(1, D))


def _tc_rest(emb, self_emb, parts, W2, b2):
    N, D = emb.shape

    def body(emb_ref, self_ref, parts_ref, w2_ref, b2_ref, out_ref):
        x = emb_ref[...]
        agg = parts_ref[0] + parts_ref[1]
        w2 = w2_ref[...]
        b2v = b2_ref[...]
        neigh = lax.dot_general(agg, w2, _DN,
                                preferred_element_type=jnp.float32) + b2v
        inter = lax.dot_general(neigh * x, w2, _DN,
                                preferred_element_type=jnp.float32) + b2v
        o = self_ref[...] + neigh + inter
        out_ref[...] = jnp.where(o >= 0, o, 0.2 * o)

    return pl.pallas_call(
        body,
        grid=(N // _BM,),
        in_specs=[
            pl.BlockSpec((_BM, D), lambda i: (i, 0)),
            pl.BlockSpec((_BM, D), lambda i: (i, 0)),
            pl.BlockSpec((NC, _BM, D), lambda i: (0, i, 0)),
            pl.BlockSpec((D, D), lambda i: (0, 0)),
            pl.BlockSpec((1, D), lambda i: (0, 0)),
        ],
        out_specs=pl.BlockSpec((_BM, D), lambda i: (i, 0)),
        out_shape=jax.ShapeDtypeStruct((N, D), jnp.float32),
    )(emb, self_emb, parts, W2, b2.reshape(1, D))


def kernel(embeddings, adj_edge_index, adj_edge_weight, W1, b1, W2, b2):
    N, D = embeddings.shape
    src = adj_edge_index[0]
    dst = adj_edge_index[1]
    Np = -(-N // (8 * NS)) * (8 * NS)
    zeros = jnp.zeros((Np, D), jnp.float32)
    parts = _sc_spmm(embeddings, src, dst, adj_edge_weight, zeros)
    self_emb = _tc_self(embeddings, W1, b1)
    return _tc_rest(embeddings, self_emb, parts, W2, b2)


# speedup vs baseline: 1.4365x; 1.0063x over previous
import functools

import jax
import jax.numpy as jnp
from jax import lax
from jax.experimental import pallas as pl
from jax.experimental.pallas import tpu as pltpu
from jax.experimental.pallas import tpu_sc as plsc

NC = 2
NS = 16
LANES = 16
NW = NC * NS

CH = 80


def _sc_spmm(emb, src, dst, w, zeros):
    N, D = emb.shape
    E = src.shape[0]
    epw = E // NW
    n_chunks = epw // CH
    Np = zeros.shape[0]
    rows_per_tile = Np // NS

    mesh = plsc.VectorSubcoreMesh(core_axis_name="c", subcore_axis_name="s")

    assert n_chunks % 2 == 1

    @functools.partial(
        pl.kernel,
        out_type=jax.ShapeDtypeStruct((NC, Np, D), jnp.float32),
        mesh=mesh,
        compiler_params=pltpu.CompilerParams(needs_layout_passes=False,
                                             use_tc_tiling_on_sc=False),
        scratch_types=[
            pltpu.VMEM_SHARED((Np, D), jnp.float32),
            pltpu.VMEM((epw,), jnp.int32),
            pltpu.VMEM((epw,), jnp.float32),
            pltpu.VMEM((CH,), jnp.int32),
            pltpu.VMEM((CH,), jnp.int32),
            pltpu.VMEM((CH, D), jnp.float32),
            pltpu.VMEM((CH, D), jnp.float32),
            pltpu.SemaphoreType.DMA,
            pltpu.SemaphoreType.DMA,
            pltpu.SemaphoreType.DMA,
            pltpu.SemaphoreType.DMA,
        ],
    )
    def spmm(emb_hbm, src_hbm, dst_hbm, w_hbm, zeros_hbm, parts_hbm,
             agg_sh, src_v, w_v, dbuf0, dbuf1, rows0, rows1,
             gsem0, gsem1, dsem0, dsem1):
        c = lax.axis_index("c")
        s = lax.axis_index("s")
        wid = s * NC + c
        r0 = s * rows_per_tile
        pltpu.sync_copy(zeros_hbm.at[pl.ds(r0, rows_per_tile)],
                        agg_sh.at[pl.ds(r0, rows_per_tile)])
        pltpu.sync_copy(src_hbm.at[wid], src_v)
        pltpu.sync_copy(w_hbm.at[wid], w_v)
        plsc.subcore_barrier()

        rows = (rows0, rows1)
        gsems = (gsem0, gsem1)
        dbufs = (dbuf0, dbuf1)
        dsems = (dsem0, dsem1)

        def issue_chunk(i, b):
            pltpu.async_copy(dst_hbm.at[wid, i], dbufs[b], dsems[b])
            pltpu.async_copy(emb_hbm.at[src_v.at[pl.ds(i * CH, CH)]],
                             rows[b], gsems[b])

        def wait_gather(b):
            pltpu.make_async_copy(emb_hbm.at[src_v.at[pl.ds(0, CH)]],
                                  rows[b], gsems[b]).wait()

        def mul_chunk(i, b):
            rbuf = rows[b]

            def group_body(g, _):
                w16 = w_v[pl.ds(i * CH + g * LANES, LANES)]
                for el in range(LANES):
                    wb = w16[el]
                    e = g * LANES + el
                    for k in range(D // LANES):
                        sl = pl.ds(k * LANES, LANES)
                        rbuf[e, sl] = rbuf[e, sl] * wb
                return 0

            lax.fori_loop(0, CH // LANES, group_body, 0)

        def scatter_chunk(b):
            pltpu.make_async_copy(dst_hbm.at[wid, 0], dbufs[b], dsems[b]).wait()
            pltpu.sync_copy(rows[b], agg_sh.at[dbufs[b]], add=True)

        issue_chunk(0, 0)

        @pl.loop(0, n_chunks - 1, step=2)
        def _(t):
            issue_chunk(t + 1, 1)
            wait_gather(0)
            mul_chunk(t, 0)
            scatter_chunk(0)
            issue_chunk(t + 2, 0)
            wait_gather(1)
            mul_chunk(t + 1, 1)
            scatter_chunk(1)

        wait_gather(0)
        mul_chunk(n_chunks - 1, 0)
        scatter_chunk(0)

        plsc.subcore_barrier()
        pltpu.sync_copy(agg_sh.at[pl.ds(r0, rows_per_tile)],
                        parts_hbm.at[c, pl.ds(r0, rows_per_tile)])

    return spmm(emb, src.reshape(NW, epw), dst.reshape(NW, n_chunks, CH),
                w.reshape(NW, epw), zeros)


def _tc_dense(emb, parts, W1, b1, W2, b2):
    N, D = emb.shape
    BM = 2000
    dn = (((1,), (1,)), ((), ()))

    def body(emb_ref, parts_ref, w1_ref, b1_ref, w2_ref, b2_ref, out_ref):
        x = emb_ref[...]
        agg = parts_ref[0] + parts_ref[1]
        w1 = w1_ref[...]
        w2 = w2_ref[...]
        b1v = b1_ref[...]
        b2v = b2_ref[...]
        self_emb = lax.dot_general(x, w1, dn, preferred_element_type=jnp.float32) + b1v
        neigh = lax.dot_general(agg, w2, dn, preferred_element_type=jnp.float32) + b2v
        inter = lax.dot_general(neigh * x, w2, dn,
                                preferred_element_type=jnp.float32) + b2v
        o = self_emb + neigh + inter
        out_ref[...] = jnp.where(o >= 0, o, 0.2 * o)

    return pl.pallas_call(
        body,
        grid=(N // BM,),
        in_specs=[
            pl.BlockSpec((BM, D), lambda i: (i, 0)),
            pl.BlockSpec((NC, BM, D), lambda i: (0, i, 0)),
            pl.BlockSpec((D, D), lambda i: (0, 0)),
            pl.BlockSpec((1, D), lambda i: (0, 0)),
            pl.BlockSpec((D, D), lambda i: (0, 0)),
            pl.BlockSpec((1, D), lambda i: (0, 0)),
        ],
        out_specs=pl.BlockSpec((BM, D), lambda i: (i, 0)),
        out_shape=jax.ShapeDtypeStruct((N, D), jnp.float32),
    )(emb, parts, W1, b1.reshape(1, D), W2, b2.reshape(1, D))


def kernel(embeddings, adj_edge_index, adj_edge_weight, W1, b1, W2, b2):
    N, D = embeddings.shape
    src = adj_edge_index[0]
    dst = adj_edge_index[1]
    Np = -(-N // (8 * NS)) * (8 * NS)
    zeros = jnp.zeros((Np, D), jnp.float32)
    parts = _sc_spmm(embeddings, src, dst, adj_edge_weight, zeros)
    return _tc_dense(embeddings, parts, W1, b1, W2, b2)
